# baseline (speedup 1.0000x reference)
import functools

import jax
import jax.numpy as jnp
from jax import lax
from jax.experimental import pallas as pl
from jax.experimental.pallas import tpu as pltpu
from jax.experimental.pallas import tpu_sc as plsc

_NC = 2
_NS = 16
_NW = _NC * _NS
_L = 16
_SG = 128


def _make_gather(batch, vocab, dim):
    b_per_w = batch // _NW
    n_sg = b_per_w // _SG
    half = vocab // 2
    mesh = plsc.VectorSubcoreMesh(core_axis_name="c", subcore_axis_name="s")

    @functools.partial(
        pl.kernel,
        mesh=mesh,
        out_type=jax.ShapeDtypeStruct((batch, dim), jnp.float32),
        scratch_types=[
            pltpu.VMEM((b_per_w,), jnp.int32),
            pltpu.VMEM((_SG,), jnp.int32),
            pltpu.VMEM((_SG,), jnp.int32),
            pltpu.VMEM((_SG, 2 * dim), jnp.float32),
            pltpu.VMEM((_SG, 2 * dim), jnp.float32),
            pltpu.VMEM((b_per_w, dim), jnp.float32),
            pltpu.SemaphoreType.DMA,
        ],
    )
    def gather_kernel(idx_hbm, pairs_a, pairs_b, out_hbm, idx_v, la_v, lb_v,
                      bufa_v, bufb_v, rows_v, sem):
        wid = lax.axis_index("s") * _NC + lax.axis_index("c")
        base = wid * b_per_w
        pltpu.sync_copy(idx_hbm.at[pl.ds(base, b_per_w)], idx_v)

        for c in range(n_sg):
            for b in range(_SG // _L):
                v = idx_v[pl.ds(c * _SG + b * _L, _L)]
                near = v < half
                zero = jnp.zeros((_L,), jnp.int32)
                la_v[pl.ds(b * _L, _L)] = jnp.where(near, v >> 1, zero)
                lb_v[pl.ds(b * _L, _L)] = jnp.where(near, zero, (v - half) >> 1)

            cpa = pltpu.async_copy(pairs_a.at[la_v], bufa_v, sem)
            cpb = pltpu.async_copy(pairs_b.at[lb_v], bufb_v, sem)
            cpa.wait()
            cpb.wait()

            def select_block(b, carry):
                v = idx_v[pl.ds(c * _SG + b * _L, _L)]
                par = v & 1
                for j in range(_L):
                    row = b * _L + j
                    near_j = v[j] < half
                    off = par[j] * dim

                    @pl.when(near_j)
                    def _():
                        for k in range(dim // _L):
                            rows_v[c * _SG + row, pl.ds(k * _L, _L)] = (
                                bufa_v[row, pl.ds(off + k * _L, _L)]
                            )

                    @pl.when(jnp.logical_not(near_j))
                    def _():
                        for k in range(dim // _L):
                            rows_v[c * _SG + row, pl.ds(k * _L, _L)] = (
                                bufb_v[row, pl.ds(off + k * _L, _L)]
                            )
                return carry

            lax.fori_loop(0, _SG // _L, select_block, 0)

        pltpu.sync_copy(rows_v, out_hbm.at[pl.ds(base, b_per_w)])

    return gather_kernel


def kernel(center_words, center_table):
    batch = center_words.shape[0]
    vocab, dim = center_table.shape
    half = vocab // 2
    idx = center_words.astype(jnp.int32)
    pairs_a = center_table[:half].reshape(half // 2, 2 * dim)
    pairs_b = center_table[half:].reshape(half // 2, 2 * dim)
    return _make_gather(batch, vocab, dim)(idx, pairs_a, pairs_b)


# speedup vs baseline: 1.4357x; 1.4357x over previous
import functools

import jax
import jax.numpy as jnp
from jax import lax
from jax.experimental import pallas as pl
from jax.experimental.pallas import tpu as pltpu
from jax.experimental.pallas import tpu_sc as plsc

_NC = 2
_NS = 16
_NW = _NC * _NS
_K = 32
_N_SC = 8192
_R = 8


def _make_sc_gather(batch, vocab, dim):
    b_per_w = batch // _NW
    n_step = b_per_w // _K
    mesh = plsc.VectorSubcoreMesh(core_axis_name="c", subcore_axis_name="s")

    @functools.partial(
        pl.kernel,
        mesh=mesh,
        out_type=jax.ShapeDtypeStruct((batch, dim), jnp.float32),
        scratch_types=[
            pltpu.VMEM((b_per_w,), jnp.int32),
            pltpu.VMEM((b_per_w, dim), jnp.float32),
            pltpu.SemaphoreType.DMA,
        ],
    )
    def gather_kernel(idx_hbm, table_hbm, out_hbm, idx_v, rows_v, sem):
        wid = lax.axis_index("s") * _NC + lax.axis_index("c")
        base = wid * b_per_w
        pltpu.sync_copy(idx_hbm.at[pl.ds(base, b_per_w)], idx_v)

        def fire(s):
            idx_vec = idx_v[pl.ds(s * _K, _K)]
            for j in range(_K):
                row = idx_vec[j]
                pltpu.async_copy(
                    table_hbm.at[pl.ds(row, 1), :],
                    rows_v.at[pl.ds(s * _K + j, 1), :],
                    sem,
                )

        def drain():
            for _ in range(_K):
                pltpu.make_async_copy(
                    table_hbm.at[pl.ds(0, 1), :],
                    rows_v.at[pl.ds(0, 1), :],
                    sem,
                ).wait()

        fire(0)

        def step(s, carry):
            fire(s)
            drain()
            return carry

        lax.fori_loop(1, n_step, step, 0)
        drain()
        pltpu.sync_copy(rows_v, out_hbm.at[pl.ds(base, b_per_w)])

    return gather_kernel


def _make_tc_gather(batch, vocab, dim):
    n_step = batch // _R

    def tc_body(idx_ref, *refs):
        in_refs = refs[:_R]
        out_ref = refs[_R]
        i = pl.program_id(0)
        for k in range(_R):
            r = idx_ref[_R * i + k] % 8
            out_ref[pl.ds(k, 1), :] = in_refs[k][pl.ds(r, 1), :]

    return pl.pallas_call(
        tc_body,
        grid_spec=pltpu.PrefetchScalarGridSpec(
            num_scalar_prefetch=1,
            grid=(n_step,),
            in_specs=[
                pl.BlockSpec(
                    (8, dim),
                    functools.partial(
                        lambda i, idx_ref, k: (idx_ref[_R * i + k] // 8, 0),
                        k=k,
                    ),
                )
                for k in range(_R)
            ],
            out_specs=pl.BlockSpec((_R, dim), lambda i, idx_ref: (i, 0)),
        ),
        out_shape=jax.ShapeDtypeStruct((batch, dim), jnp.float32),
    )


def kernel(center_words, center_table):
    batch = center_words.shape[0]
    vocab, dim = center_table.shape
    idx = center_words.astype(jnp.int32)
    idx_sc = idx[:_N_SC]
    idx_tc = idx[_N_SC:]
    n_tc = batch - _N_SC

    out_sc = _make_sc_gather(_N_SC, vocab, dim)(idx_sc, center_table)

    tc_call = _make_tc_gather(n_tc, vocab, dim)
    out_tc = tc_call(idx_tc, *([center_table] * _R))

    return jnp.concatenate([out_sc, out_tc], axis=0)


# speedup vs baseline: 3.4670x; 2.4148x over previous
import functools

import jax
import jax.numpy as jnp
from jax import lax
from jax.experimental import pallas as pl
from jax.experimental.pallas import tpu as pltpu
from jax.experimental.pallas import tpu_sc as plsc

_NC = 2
_NS = 16
_NW = _NC * _NS
_K = 32


def _make_sc_gather(batch, vocab, dim):
    b_per_w = batch // _NW
    n_step = b_per_w // _K
    mesh = plsc.VectorSubcoreMesh(core_axis_name="c", subcore_axis_name="s")

    @functools.partial(
        pl.kernel,
        mesh=mesh,
        out_type=jax.ShapeDtypeStruct((batch, dim), jnp.float32),
        scratch_types=[
            pltpu.VMEM((b_per_w,), jnp.int32),
            pltpu.VMEM((b_per_w, dim), jnp.float32),
            pltpu.SemaphoreType.DMA,
        ],
        compiler_params=pltpu.CompilerParams(skip_device_barrier=True),
    )
    def gather_kernel(idx_hbm, table_hbm, out_hbm, idx_v, rows_v, sem):
        wid = lax.axis_index("s") * _NC + lax.axis_index("c")
        base = wid * b_per_w
        pltpu.sync_copy(idx_hbm.at[pl.ds(base, b_per_w)], idx_v)

        def fire(s):
            idx_vec = idx_v[pl.ds(s * _K, _K)]
            for j in range(_K):
                row = idx_vec[j]
                pltpu.async_copy(
                    table_hbm.at[pl.ds(row, 1), :],
                    rows_v.at[pl.ds(s * _K + j, 1), :],
                    sem,
                )

        def drain():
            for _ in range(_K):
                pltpu.make_async_copy(
                    table_hbm.at[pl.ds(0, 1), :],
                    rows_v.at[pl.ds(0, 1), :],
                    sem,
                ).wait()

        fire(0)

        def step(s, carry):
            fire(s)
            drain()
            return carry

        lax.fori_loop(1, n_step, step, 0)
        drain()
        pltpu.sync_copy(rows_v, out_hbm.at[pl.ds(base, b_per_w)])

    return gather_kernel


def kernel(center_words, center_table):
    batch = center_words.shape[0]
    vocab, dim = center_table.shape
    idx = center_words.astype(jnp.int32)
    return _make_sc_gather(batch, vocab, dim)(idx, center_table)
